# Initial kernel scaffold; baseline (speedup 1.0000x reference)
import jax
import jax.numpy as jnp
from jax.experimental import pallas as pl


def kernel(weights, prev_mask, epoch):
    raise NotImplementedError("write your pallas kernel here")


# speedup vs baseline: 160.5393x; 160.5393x over previous
import functools

import jax
import jax.numpy as jnp
from jax.experimental import pallas as pl


def _mask_block_kernel(w_ref, m_ref, out_ref, *, drop_n):
    w = w_ref[...] * m_ref[...]
    u = jax.lax.bitcast_convert_type(jnp.abs(w), jnp.int32)
    rb = u.shape[0]

    def body(i, carry):
        prefix, k = carry
        s = 30 - i
        cand = prefix | (1 << s)
        c = jnp.sum((u >> s) == (cand >> s), axis=1, keepdims=True,
                    dtype=jnp.int32)
        take = k <= c
        prefix = jnp.where(take, cand, prefix)
        k = jnp.where(take, k, k - c)
        return prefix, k

    prefix0 = jnp.zeros((rb, 1), jnp.int32)
    k0 = jnp.full((rb, 1), drop_n, jnp.int32)
    t, _ = jax.lax.fori_loop(0, 31, body, (prefix0, k0))
    out_ref[...] = (u >= t).astype(jnp.float32)


def kernel(weights, prev_mask, epoch):
    del epoch
    R, D = weights.shape
    drop_n = int(round(D - 0.1 * D))
    rb = min(R, 16)
    grid = (R // rb,)
    return pl.pallas_call(
        functools.partial(_mask_block_kernel, drop_n=drop_n),
        grid=grid,
        in_specs=[
            pl.BlockSpec((rb, D), lambda i: (i, 0)),
            pl.BlockSpec((rb, D), lambda i: (i, 0)),
        ],
        out_specs=pl.BlockSpec((rb, D), lambda i: (i, 0)),
        out_shape=jax.ShapeDtypeStruct((R, D), jnp.float32),
    )(weights, prev_mask)
